# Initial kernel scaffold; baseline (speedup 1.0000x reference)
import functools

import jax
import jax.numpy as jnp
from jax import lax
from jax.experimental import pallas as pl
from jax.experimental.pallas import tpu as pltpu
from jax.experimental.pallas import tpu_sc as plsc

D = 64
L = 16
NC, NS = 2, 16
NW = NC * NS
N = 64 * 64 * 64
PER_W = N // NW
CHUNK = 512
NCHUNK = PER_W // CHUNK
IDXW = 128
IDXR = CHUNK // IDXW
EPS = 1e-12

_mesh = plsc.VectorSubcoreMesh(core_axis_name="c", subcore_axis_name="s")


@functools.partial(
    pl.kernel,
    mesh=_mesh,
    out_type=jax.ShapeDtypeStruct((N, D), jnp.float32),
    scratch_types=[
        pltpu.VMEM((IDXR, IDXW), jnp.int32),
        pltpu.VMEM((IDXR, IDXW), jnp.int32),
        pltpu.VMEM((CHUNK, D), jnp.float32),
        pltpu.VMEM((CHUNK, D), jnp.float32),
        pltpu.VMEM((CHUNK, D), jnp.float32),
        pltpu.VMEM((D,), jnp.float32),
        pltpu.VMEM((D,), jnp.float32),
        pltpu.SemaphoreType.DMA,
        pltpu.SemaphoreType.DMA,
    ],
)
def _emb_ln(ids_hbm, pids_hbm, wt_hbm, pt_hbm, lw_hbm, lb_hbm, out_hbm,
            idx_v, pidx_v, wrows, prows, outb, lw_v, lb_v, semw, semp):
    wid = lax.axis_index("s") * NC + lax.axis_index("c")
    pltpu.sync_copy(lw_hbm, lw_v)
    pltpu.sync_copy(lb_hbm, lb_v)
    lw = [lw_v[pl.ds(k * L, L)] for k in range(D // L)]
    lb = [lb_v[pl.ds(k * L, L)] for k in range(D // L)]

    def chunk_body(c, carry):
        tbase = wid * PER_W + c * CHUNK
        rbase = wid * (PER_W // IDXW) + c * IDXR
        pltpu.sync_copy(ids_hbm.at[pl.ds(rbase, IDXR)], idx_v)
        pltpu.sync_copy(pids_hbm.at[pl.ds(rbase, IDXR)], pidx_v)
        copies = []
        for j in range(IDXR):
            copies.append(pltpu.async_copy(
                wt_hbm.at[idx_v.at[j]], wrows.at[pl.ds(j * IDXW, IDXW)], semw))
            copies.append(pltpu.async_copy(
                pt_hbm.at[pidx_v.at[j]], prows.at[pl.ds(j * IDXW, IDXW)], semp))
        for cp in copies:
            cp.wait()

        def tok(t, carry2):
            xs = [wrows[t, pl.ds(k * L, L)] + prows[t, pl.ds(k * L, L)]
                  for k in range(D // L)]
            s = (xs[0] + xs[1]) + (xs[2] + xs[3])
            q = (xs[0] * xs[0] + xs[1] * xs[1]) + (xs[2] * xs[2] + xs[3] * xs[3])
            u = jnp.sum(s) * (1.0 / D)
            var = jnp.sum(q) * (1.0 / D) - u * u
            vv = jnp.maximum(var, 0.0) + EPS
            seed = jnp.int32(0x5F3759DF) - lax.shift_right_arithmetic(
                lax.bitcast_convert_type(vv, jnp.int32), 1)
            y = lax.bitcast_convert_type(seed, jnp.float32)
            y = y * (1.5 - 0.5 * vv * y * y)
            y = y * (1.5 - 0.5 * vv * y * y)
            y = y * (1.5 - 0.5 * vv * y * y)
            for k in range(D // L):
                outb[t, pl.ds(k * L, L)] = (xs[k] - u) * y * lw[k] + lb[k]
            return carry2

        lax.fori_loop(0, CHUNK, tok, 0, unroll=2)
        pltpu.sync_copy(outb, out_hbm.at[pl.ds(tbase, CHUNK)])
        return carry

    lax.fori_loop(0, NCHUNK, chunk_body, 0)


def kernel(input_ids, position_ids, word_table, pos_table, ln_weight, ln_bias):
    ids = input_ids.reshape(N // IDXW, IDXW)
    pids = position_ids.reshape(N // IDXW, IDXW)
    out = _emb_ln(ids, pids, word_table, pos_table, ln_weight, ln_bias)
    return out.reshape(*input_ids.shape, D)


# speedup vs baseline: 1.1786x; 1.1786x over previous
import functools

import jax
import jax.numpy as jnp
from jax import lax
from jax.experimental import pallas as pl
from jax.experimental.pallas import tpu as pltpu
from jax.experimental.pallas import tpu_sc as plsc

D = 64
L = 16
NC, NS = 2, 16
NW = NC * NS
N = 64 * 64 * 64
PER_W = N // NW
CHUNK = 512
NCHUNK = PER_W // CHUNK
IDXW = 128
IDXR = CHUNK // IDXW
EPS = 1e-12

_mesh = plsc.VectorSubcoreMesh(core_axis_name="c", subcore_axis_name="s")


@functools.partial(
    pl.kernel,
    mesh=_mesh,
    out_type=jax.ShapeDtypeStruct((N, D), jnp.float32),
    scratch_types=[
        pltpu.VMEM((IDXR, IDXW), jnp.int32),
        pltpu.VMEM((IDXR, IDXW), jnp.int32),
        pltpu.VMEM((CHUNK, D), jnp.float32),
        pltpu.VMEM((CHUNK, D), jnp.float32),
        pltpu.VMEM((CHUNK, D), jnp.float32),
        pltpu.VMEM((D,), jnp.float32),
        pltpu.VMEM((D,), jnp.float32),
        pltpu.SemaphoreType.DMA,
        pltpu.SemaphoreType.DMA,
    ],
    compiler_params=pltpu.CompilerParams(use_tc_tiling_on_sc=False),
)
def _emb_ln(ids_hbm, pids_hbm, wt_hbm, pt_hbm, lw_hbm, lb_hbm, out_hbm,
            idx_v, pidx_v, wrows, prows, outb, lw_v, lb_v, semw, semp):
    wid = lax.axis_index("s") * NC + lax.axis_index("c")
    pltpu.sync_copy(lw_hbm, lw_v)
    pltpu.sync_copy(lb_hbm, lb_v)
    lw = [lw_v[pl.ds(k * L, L)] for k in range(D // L)]
    lb = [lb_v[pl.ds(k * L, L)] for k in range(D // L)]
    lanes = lax.iota(jnp.int32, L)
    perms = [lanes ^ m for m in (8, 4, 2, 1)]

    _gdn = lax.GatherDimensionNumbers(
        offset_dims=(), collapsed_slice_dims=(0,), start_index_map=(0,))

    def hsum(v):
        for p in perms:
            v = v + lax.gather(v, p[:, None], _gdn, (1,),
                               mode=lax.GatherScatterMode.PROMISE_IN_BOUNDS)
        return v

    def chunk_body(c, carry):
        tbase = wid * PER_W + c * CHUNK
        rbase = wid * (PER_W // IDXW) + c * IDXR
        pltpu.sync_copy(ids_hbm.at[pl.ds(rbase, IDXR)], idx_v)
        pltpu.sync_copy(pids_hbm.at[pl.ds(rbase, IDXR)], pidx_v)
        copies = []
        for j in range(IDXR):
            copies.append(pltpu.async_copy(
                wt_hbm.at[idx_v.at[j]], wrows.at[pl.ds(j * IDXW, IDXW)], semw))
            copies.append(pltpu.async_copy(
                pt_hbm.at[pidx_v.at[j]], prows.at[pl.ds(j * IDXW, IDXW)], semp))
        for cp in copies:
            cp.wait()

        def tok(t, carry2):
            xs = [wrows[t, pl.ds(k * L, L)] + prows[t, pl.ds(k * L, L)]
                  for k in range(D // L)]
            s = (xs[0] + xs[1]) + (xs[2] + xs[3])
            q = (xs[0] * xs[0] + xs[1] * xs[1]) + (xs[2] * xs[2] + xs[3] * xs[3])
            u = hsum(s) * (1.0 / D)
            var = hsum(q) * (1.0 / D) - u * u
            vv = jnp.maximum(var, 0.0) + EPS
            seed = jnp.int32(0x5F3759DF) - lax.shift_right_arithmetic(
                lax.bitcast_convert_type(vv, jnp.int32), 1)
            y = lax.bitcast_convert_type(seed, jnp.float32)
            y = y * (1.5 - 0.5 * vv * y * y)
            y = y * (1.5 - 0.5 * vv * y * y)
            y = y * (1.5 - 0.5 * vv * y * y)
            for k in range(D // L):
                outb[t, pl.ds(k * L, L)] = (xs[k] - u) * y * lw[k] + lb[k]
            return carry2

        lax.fori_loop(0, CHUNK, tok, 0, unroll=2)
        pltpu.sync_copy(outb, out_hbm.at[pl.ds(tbase, CHUNK)])
        return carry

    lax.fori_loop(0, NCHUNK, chunk_body, 0)


def kernel(input_ids, position_ids, word_table, pos_table, ln_weight, ln_bias):
    ids = input_ids.reshape(N // IDXW, IDXW)
    pids = position_ids.reshape(N // IDXW, IDXW)
    out = _emb_ln(ids, pids, word_table, pos_table, ln_weight, ln_bias)
    return out.reshape(*input_ids.shape, D)


# speedup vs baseline: 1.2188x; 1.0342x over previous
import functools

import jax
import jax.numpy as jnp
from jax import lax
from jax.experimental import pallas as pl
from jax.experimental.pallas import tpu as pltpu
from jax.experimental.pallas import tpu_sc as plsc

D = 64
L = 16
NC, NS = 2, 16
NW = NC * NS
N = 64 * 64 * 64
PER_W = N // NW
CHUNK = 512
NCHUNK = PER_W // CHUNK
IDXW = 128
IDXR = CHUNK // IDXW
EPS = 1e-12

_mesh = plsc.VectorSubcoreMesh(core_axis_name="c", subcore_axis_name="s")


@functools.partial(
    pl.kernel,
    mesh=_mesh,
    out_type=jax.ShapeDtypeStruct((N, D), jnp.float32),
    scratch_types=[
        pltpu.VMEM((IDXR, IDXW), jnp.int32),
        pltpu.VMEM((IDXR, IDXW), jnp.int32),
        pltpu.VMEM((CHUNK, D), jnp.float32),
        pltpu.VMEM((CHUNK, D), jnp.float32),
        pltpu.VMEM((CHUNK, D), jnp.float32),
        pltpu.VMEM((D,), jnp.float32),
        pltpu.VMEM((D,), jnp.float32),
        pltpu.SemaphoreType.DMA,
        pltpu.SemaphoreType.DMA,
    ],
    compiler_params=pltpu.CompilerParams(use_tc_tiling_on_sc=False),
)
def _emb_ln(ids_hbm, pids_hbm, wt_hbm, pt_hbm, lw_hbm, lb_hbm, out_hbm,
            idx_v, pidx_v, wrows, prows, outb, lw_v, lb_v, semw, semp):
    wid = lax.axis_index("s") * NC + lax.axis_index("c")
    pltpu.sync_copy(lw_hbm, lw_v)
    pltpu.sync_copy(lb_hbm, lb_v)
    lw = [lw_v[pl.ds(k * L, L)] for k in range(D // L)]
    lb = [lb_v[pl.ds(k * L, L)] for k in range(D // L)]
    lanes = lax.iota(jnp.int32, L)
    perms = [lanes ^ m for m in (8, 4, 2, 1)]

    _gdn = lax.GatherDimensionNumbers(
        offset_dims=(), collapsed_slice_dims=(0,), start_index_map=(0,))

    def hsum(v):
        for p in perms:
            v = v + lax.gather(v, p[:, None], _gdn, (1,),
                               mode=lax.GatherScatterMode.PROMISE_IN_BOUNDS)
        return v

    def chunk_body(c, carry):
        tbase = wid * PER_W + c * CHUNK
        rbase = wid * (PER_W // IDXW) + c * IDXR
        pltpu.sync_copy(ids_hbm.at[pl.ds(rbase, IDXR)], idx_v)
        pltpu.sync_copy(pids_hbm.at[pl.ds(rbase, IDXR)], pidx_v)
        copies = []
        for j in range(IDXR):
            copies.append(pltpu.async_copy(
                wt_hbm.at[idx_v.at[j]], wrows.at[pl.ds(j * IDXW, IDXW)], semw))
            copies.append(pltpu.async_copy(
                pt_hbm.at[pidx_v.at[j]], prows.at[pl.ds(j * IDXW, IDXW)], semp))
        for cp in copies:
            cp.wait()

        def tok(t, carry2):
            xs = [wrows[t, pl.ds(k * L, L)] + prows[t, pl.ds(k * L, L)]
                  for k in range(D // L)]
            s = (xs[0] + xs[1]) + (xs[2] + xs[3])
            q = (xs[0] * xs[0] + xs[1] * xs[1]) + (xs[2] * xs[2] + xs[3] * xs[3])
            u = hsum(s) * (1.0 / D)
            var = hsum(q) * (1.0 / D) - u * u
            vv = jnp.maximum(var, 0.0) + EPS
            seed = jnp.int32(0x5F3759DF) - lax.shift_right_arithmetic(
                lax.bitcast_convert_type(vv, jnp.int32), 1)
            y = lax.bitcast_convert_type(seed, jnp.float32)
            y = y * (1.5 - 0.5 * vv * y * y)
            y = y * (1.5 - 0.5 * vv * y * y)
            for k in range(D // L):
                outb[t, pl.ds(k * L, L)] = (xs[k] - u) * y * lw[k] + lb[k]
            return carry2

        lax.fori_loop(0, CHUNK, tok, 0, unroll=8)
        pltpu.sync_copy(outb, out_hbm.at[pl.ds(tbase, CHUNK)])
        return carry

    lax.fori_loop(0, NCHUNK, chunk_body, 0)


def kernel(input_ids, position_ids, word_table, pos_table, ln_weight, ln_bias):
    ids = input_ids.reshape(N // IDXW, IDXW)
    pids = position_ids.reshape(N // IDXW, IDXW)
    out = _emb_ln(ids, pids, word_table, pos_table, ln_weight, ln_bias)
    return out.reshape(*input_ids.shape, D)


# speedup vs baseline: 1.2376x; 1.0154x over previous
import functools

import jax
import jax.numpy as jnp
from jax import lax
from jax.experimental import pallas as pl
from jax.experimental.pallas import tpu as pltpu
from jax.experimental.pallas import tpu_sc as plsc

D = 64
L = 16
NC, NS = 2, 16
NW = NC * NS
N = 64 * 64 * 64
PER_W = N // NW
CHUNK = 256
NCHUNK = PER_W // CHUNK
IDXW = 128
IDXR = CHUNK // IDXW
MAXPOS = 512
EPS = 1e-12

_mesh = plsc.VectorSubcoreMesh(core_axis_name="c", subcore_axis_name="s")


@functools.partial(
    pl.kernel,
    mesh=_mesh,
    out_type=jax.ShapeDtypeStruct((N, D), jnp.float32),
    scratch_types=[
        pltpu.VMEM((2, IDXR, IDXW), jnp.int32),
        pltpu.VMEM((2, CHUNK), jnp.int32),
        pltpu.VMEM((2, CHUNK, D), jnp.float32),
        pltpu.VMEM((2, CHUNK, D), jnp.float32),
        pltpu.VMEM((MAXPOS, D), jnp.float32),
        pltpu.VMEM((D,), jnp.float32),
        pltpu.VMEM((D,), jnp.float32),
        pltpu.SemaphoreType.DMA,
        pltpu.SemaphoreType.DMA,
        pltpu.SemaphoreType.DMA,
        pltpu.SemaphoreType.DMA,
    ],
    compiler_params=pltpu.CompilerParams(
        use_tc_tiling_on_sc=False, needs_layout_passes=False),
)
def _emb_ln(ids_hbm, pids_hbm, wt_hbm, pt_hbm, lw_hbm, lb_hbm, out_hbm,
            idx_v, pidx_v, wrows, outb, pt_v, lw_v, lb_v,
            semw0, semw1, semo0, semo1):
    wid = lax.axis_index("s") * NC + lax.axis_index("c")
    tok0 = wid * PER_W
    row0 = wid * (PER_W // IDXW)
    semw = (semw0, semw1)
    semo = (semo0, semo1)

    pltpu.sync_copy(pt_hbm, pt_v)
    pltpu.sync_copy(lw_hbm, lw_v)
    pltpu.sync_copy(lb_hbm, lb_v)
    lw = [lw_v[pl.ds(k * L, L)] for k in range(D // L)]
    lb = [lb_v[pl.ds(k * L, L)] for k in range(D // L)]
    lanes = lax.iota(jnp.int32, L)
    perms = [lanes ^ m for m in (8, 4, 2, 1)]
    cols = [k * L + lanes for k in range(D // L)]

    _gdn = lax.GatherDimensionNumbers(
        offset_dims=(), collapsed_slice_dims=(0,), start_index_map=(0,))

    def hsum(v):
        for p in perms:
            v = v + lax.gather(v, p[:, None], _gdn, (1,),
                               mode=lax.GatherScatterMode.PROMISE_IN_BOUNDS)
        return v

    def stage_and_fire(c, b):
        pltpu.sync_copy(ids_hbm.at[pl.ds(row0 + c * IDXR, IDXR)], idx_v.at[b])
        pltpu.sync_copy(pids_hbm.at[pl.ds(tok0 + c * CHUNK, CHUNK)],
                        pidx_v.at[b])
        for j in range(IDXR):
            pltpu.async_copy(wt_hbm.at[idx_v.at[b, j]],
                             wrows.at[b, pl.ds(j * IDXW, IDXW)], semw[b])

    def wait_gather(b):
        pltpu.make_async_copy(wt_hbm.at[pl.ds(0, CHUNK)], wrows.at[b],
                              semw[b]).wait()

    def wait_out(b):
        pltpu.make_async_copy(outb.at[b], out_hbm.at[pl.ds(0, CHUNK)],
                              semo[b]).wait()

    def compute(c, b):
        def tok(t, carry):
            pid = plsc.load_gather(pidx_v.at[b], [jnp.full((L,), t, jnp.int32)])
            xs = []
            for k in range(D // L):
                xs.append(wrows[b, t, pl.ds(k * L, L)]
                          + plsc.load_gather(pt_v, [pid, cols[k]]))
            s = (xs[0] + xs[1]) + (xs[2] + xs[3])
            q = (xs[0] * xs[0] + xs[1] * xs[1]) + (xs[2] * xs[2] + xs[3] * xs[3])
            u = hsum(s) * (1.0 / D)
            var = hsum(q) * (1.0 / D) - u * u
            vv = jnp.maximum(var, 0.0) + EPS
            seed = jnp.int32(0x5F3759DF) - lax.shift_right_arithmetic(
                lax.bitcast_convert_type(vv, jnp.int32), 1)
            y = lax.bitcast_convert_type(seed, jnp.float32)
            y = y * (1.5 - 0.5 * vv * y * y)
            y = y * (1.5 - 0.5 * vv * y * y)
            for k in range(D // L):
                outb[b, t, pl.ds(k * L, L)] = (xs[k] - u) * y * lw[k] + lb[k]
            return carry

        lax.fori_loop(0, CHUNK, tok, 0, unroll=8)
        pltpu.async_copy(outb.at[b],
                         out_hbm.at[pl.ds(tok0 + c * CHUNK, CHUNK)], semo[b])

    stage_and_fire(0, 0)

    def pair_body(c2, carry):
        for b in range(2):
            c = c2 * 2 + b
            if b == 0:
                stage_and_fire(c + 1, 1)
            else:
                @pl.when(c2 < NCHUNK // 2 - 1)
                def _():
                    stage_and_fire(c + 1, 0)

            wait_gather(b)

            @pl.when(c2 >= 1)
            def _():
                wait_out(b)

            compute(c, b)
        return carry

    lax.fori_loop(0, NCHUNK // 2, pair_body, 0)
    wait_out(0)
    wait_out(1)


def kernel(input_ids, position_ids, word_table, pos_table, ln_weight, ln_bias):
    ids = input_ids.reshape(N // IDXW, IDXW)
    pids = position_ids.reshape(N)
    out = _emb_ln(ids, pids, word_table, pos_table, ln_weight, ln_bias)
    return out.reshape(*input_ids.shape, D)


# speedup vs baseline: 1.5573x; 1.2584x over previous
import functools

import jax
import jax.numpy as jnp
from jax import lax
from jax.experimental import pallas as pl
from jax.experimental.pallas import tpu as pltpu
from jax.experimental.pallas import tpu_sc as plsc

D = 64
L = 16
NC, NS = 2, 16
NW = NC * NS
N = 64 * 64 * 64
PER_W = N // NW
CHUNK = 256
NCHUNK = PER_W // CHUNK
IDXW = 128
IDXR = CHUNK // IDXW
MAXPOS = 512
EPS = 1e-12

_mesh = plsc.VectorSubcoreMesh(core_axis_name="c", subcore_axis_name="s")


@functools.partial(
    pl.kernel,
    mesh=_mesh,
    out_type=jax.ShapeDtypeStruct((N, D), jnp.float32),
    scratch_types=[
        pltpu.VMEM((2, IDXR, IDXW), jnp.int32),
        pltpu.VMEM((2, CHUNK), jnp.int32),
        pltpu.VMEM((2, CHUNK, D), jnp.float32),
        pltpu.VMEM((2, CHUNK, D), jnp.float32),
        pltpu.VMEM((MAXPOS, D), jnp.float32),
        pltpu.VMEM((D,), jnp.float32),
        pltpu.VMEM((D,), jnp.float32),
        pltpu.SemaphoreType.DMA,
        pltpu.SemaphoreType.DMA,
        pltpu.SemaphoreType.DMA,
        pltpu.SemaphoreType.DMA,
    ],
    compiler_params=pltpu.CompilerParams(
        use_tc_tiling_on_sc=False, needs_layout_passes=False),
)
def _emb_ln(ids_hbm, pids_hbm, wt_hbm, pt_hbm, lw_hbm, lb_hbm, out_hbm,
            idx_v, pidx_v, wrows, outb, pt_v, lw_v, lb_v,
            semw0, semw1, semo0, semo1):
    wid = lax.axis_index("s") * NC + lax.axis_index("c")
    tok0 = wid * PER_W
    row0 = wid * (PER_W // IDXW)
    semw = (semw0, semw1)
    semo = (semo0, semo1)

    pltpu.sync_copy(pt_hbm, pt_v)
    pltpu.sync_copy(lw_hbm, lw_v)
    pltpu.sync_copy(lb_hbm, lb_v)
    lw = [lw_v[pl.ds(k * L, L)] for k in range(D // L)]
    lb = [lb_v[pl.ds(k * L, L)] for k in range(D // L)]
    lanes = lax.iota(jnp.int32, L)
    perms = [lanes ^ m for m in (8, 4, 2, 1)]
    cols = [k * L + lanes for k in range(D // L)]

    _gdn = lax.GatherDimensionNumbers(
        offset_dims=(), collapsed_slice_dims=(0,), start_index_map=(0,))

    def hsum(v):
        for p in perms:
            v = v + lax.gather(v, p[:, None], _gdn, (1,),
                               mode=lax.GatherScatterMode.PROMISE_IN_BOUNDS)
        return v

    def stage_and_fire(c, b):
        pltpu.sync_copy(ids_hbm.at[pl.ds(row0 + c * IDXR, IDXR)], idx_v.at[b])
        pltpu.sync_copy(pids_hbm.at[pl.ds(tok0 + c * CHUNK, CHUNK)],
                        pidx_v.at[b])
        for j in range(IDXR):
            pltpu.async_copy(wt_hbm.at[idx_v.at[b, j]],
                             wrows.at[b, pl.ds(j * IDXW, IDXW)], semw[b])

    def wait_gather(b):
        pltpu.make_async_copy(wt_hbm.at[pl.ds(0, CHUNK)], wrows.at[b],
                              semw[b]).wait()

    def wait_out(b):
        pltpu.make_async_copy(outb.at[b], out_hbm.at[pl.ds(0, CHUNK)],
                              semo[b]).wait()

    def compute(c, b):
        @plsc.parallel_loop(0, CHUNK, unroll=8)
        def tok(t):
            pid = plsc.load_gather(pidx_v.at[b], [jnp.full((L,), t, jnp.int32)])
            xs = []
            for k in range(D // L):
                xs.append(wrows[b, t, pl.ds(k * L, L)]
                          + plsc.load_gather(pt_v, [pid, cols[k]]))
            s = (xs[0] + xs[1]) + (xs[2] + xs[3])
            q = (xs[0] * xs[0] + xs[1] * xs[1]) + (xs[2] * xs[2] + xs[3] * xs[3])
            u = hsum(s) * (1.0 / D)
            var = hsum(q) * (1.0 / D) - u * u
            vv = jnp.maximum(var, 0.0) + EPS
            seed = jnp.int32(0x5F3759DF) - lax.shift_right_arithmetic(
                lax.bitcast_convert_type(vv, jnp.int32), 1)
            y = lax.bitcast_convert_type(seed, jnp.float32)
            y = y * (1.5 - 0.5 * vv * y * y)
            y = y * (1.5 - 0.5 * vv * y * y)
            for k in range(D // L):
                outb[b, t, pl.ds(k * L, L)] = (xs[k] - u) * y * lw[k] + lb[k]

        pltpu.async_copy(outb.at[b],
                         out_hbm.at[pl.ds(tok0 + c * CHUNK, CHUNK)], semo[b])

    stage_and_fire(0, 0)

    def pair_body(c2, carry):
        for b in range(2):
            c = c2 * 2 + b
            if b == 0:
                stage_and_fire(c + 1, 1)
            else:
                @pl.when(c2 < NCHUNK // 2 - 1)
                def _():
                    stage_and_fire(c + 1, 0)

            wait_gather(b)

            @pl.when(c2 >= 1)
            def _():
                wait_out(b)

            compute(c, b)
        return carry

    lax.fori_loop(0, NCHUNK // 2, pair_body, 0)
    wait_out(0)
    wait_out(1)


def kernel(input_ids, position_ids, word_table, pos_table, ln_weight, ln_bias):
    ids = input_ids.reshape(N // IDXW, IDXW)
    pids = position_ids.reshape(N)
    out = _emb_ln(ids, pids, word_table, pos_table, ln_weight, ln_bias)
    return out.reshape(*input_ids.shape, D)


# speedup vs baseline: 1.6167x; 1.0381x over previous
import functools

import jax
import jax.numpy as jnp
from jax import lax
from jax.experimental import pallas as pl
from jax.experimental.pallas import tpu as pltpu
from jax.experimental.pallas import tpu_sc as plsc

D = 64
L = 16
NC, NS = 2, 16
NW = NC * NS
N = 64 * 64 * 64
PER_W = N // NW
CHUNK = 128
NCHUNK = PER_W // CHUNK
MAXPOS = 512
EPS = 1e-12

_mesh = plsc.VectorSubcoreMesh(core_axis_name="c", subcore_axis_name="s")


@functools.partial(
    pl.kernel,
    mesh=_mesh,
    out_type=jax.ShapeDtypeStruct((N, D), jnp.float32),
    scratch_types=[
        pltpu.VMEM((2, CHUNK), jnp.int32),
        pltpu.VMEM((2, CHUNK), jnp.int32),
        pltpu.VMEM((2, 1, CHUNK), jnp.int32),
        pltpu.VMEM((2, CHUNK, 2 * D), jnp.float32),
        pltpu.VMEM((2, CHUNK, D), jnp.float32),
        pltpu.VMEM((MAXPOS // 2, 2 * D), jnp.float32),
        pltpu.VMEM((D,), jnp.float32),
        pltpu.VMEM((D,), jnp.float32),
        pltpu.SemaphoreType.DMA,
        pltpu.SemaphoreType.DMA,
        pltpu.SemaphoreType.DMA,
        pltpu.SemaphoreType.DMA,
    ],
    compiler_params=pltpu.CompilerParams(
        use_tc_tiling_on_sc=True, needs_layout_passes=False),
)
def _emb_ln(ids_hbm, pids_hbm, wt_hbm, pt_hbm, lw_hbm, lb_hbm, out_hbm,
            wids_v, pidx_v, idx_v, wrows, outb, pt_v, lw_v, lb_v,
            semw0, semw1, semo0, semo1):
    wid = lax.axis_index("s") * NC + lax.axis_index("c")
    tok0 = wid * PER_W
    semw = (semw0, semw1)
    semo = (semo0, semo1)

    pltpu.sync_copy(pt_hbm, pt_v)
    pltpu.sync_copy(lw_hbm, lw_v)
    pltpu.sync_copy(lb_hbm, lb_v)
    lw = [lw_v[pl.ds(k * L, L)] for k in range(D // L)]
    lb = [lb_v[pl.ds(k * L, L)] for k in range(D // L)]
    lanes = lax.iota(jnp.int32, L)
    perms = [lanes ^ m for m in (8, 4, 2, 1)]
    cols = [k * L + lanes for k in range(D // L)]

    _gdn = lax.GatherDimensionNumbers(
        offset_dims=(), collapsed_slice_dims=(0,), start_index_map=(0,))

    def hsum(v):
        for p in perms:
            v = v + lax.gather(v, p[:, None], _gdn, (1,),
                               mode=lax.GatherScatterMode.PROMISE_IN_BOUNDS)
        return v

    def stage_and_fire(c, b):
        pltpu.sync_copy(ids_hbm.at[pl.ds(tok0 + c * CHUNK, CHUNK)],
                        wids_v.at[b])
        pltpu.sync_copy(pids_hbm.at[pl.ds(tok0 + c * CHUNK, CHUNK)],
                        pidx_v.at[b])
        for g in range(CHUNK // L):
            idx_v[b, 0, pl.ds(g * L, L)] = lax.shift_right_logical(
                wids_v[b, pl.ds(g * L, L)], 1)
        pltpu.async_copy(wt_hbm.at[idx_v.at[b, 0]], wrows.at[b], semw[b])

    def wait_gather(b):
        pltpu.make_async_copy(wt_hbm.at[pl.ds(0, CHUNK)], wrows.at[b],
                              semw[b]).wait()

    def wait_out(b):
        pltpu.make_async_copy(outb.at[b], out_hbm.at[pl.ds(0, CHUNK)],
                              semo[b]).wait()

    def compute(c, b):
        @plsc.parallel_loop(0, CHUNK, unroll=8)
        def tok(t):
            tb = jnp.full((L,), t, jnp.int32)
            wid_b = plsc.load_gather(wids_v.at[b], [tb])
            pid_b = plsc.load_gather(pidx_v.at[b], [tb])
            wcol = lax.shift_left(wid_b & 1, 6)
            prow = lax.shift_right_logical(pid_b, 1)
            pcol = lax.shift_left(pid_b & 1, 6)
            xs = []
            for k in range(D // L):
                w_k = plsc.load_gather(wrows.at[b], [tb, wcol + cols[k]])
                p_k = plsc.load_gather(pt_v, [prow, pcol + cols[k]])
                xs.append(w_k + p_k)
            s = (xs[0] + xs[1]) + (xs[2] + xs[3])
            q = (xs[0] * xs[0] + xs[1] * xs[1]) + (xs[2] * xs[2] + xs[3] * xs[3])
            u = hsum(s) * (1.0 / D)
            var = hsum(q) * (1.0 / D) - u * u
            vv = jnp.maximum(var, 0.0) + EPS
            seed = jnp.int32(0x5F3759DF) - lax.shift_right_arithmetic(
                lax.bitcast_convert_type(vv, jnp.int32), 1)
            y = lax.bitcast_convert_type(seed, jnp.float32)
            y = y * (1.5 - 0.5 * vv * y * y)
            y = y * (1.5 - 0.5 * vv * y * y)
            for k in range(D // L):
                outb[b, t, pl.ds(k * L, L)] = (xs[k] - u) * y * lw[k] + lb[k]

        pltpu.async_copy(outb.at[b],
                         out_hbm.at[pl.ds(tok0 + c * CHUNK, CHUNK)], semo[b])

    stage_and_fire(0, 0)

    def pair_body(c2, carry):
        for b in range(2):
            c = c2 * 2 + b
            if b == 0:
                stage_and_fire(c + 1, 1)
            else:
                @pl.when(c2 < NCHUNK // 2 - 1)
                def _():
                    stage_and_fire(c + 1, 0)

            wait_gather(b)

            @pl.when(c2 >= 1)
            def _():
                wait_out(b)

            compute(c, b)
        return carry

    lax.fori_loop(0, NCHUNK // 2, pair_body, 0)
    wait_out(0)
    wait_out(1)


def kernel(input_ids, position_ids, word_table, pos_table, ln_weight, ln_bias):
    ids = input_ids.reshape(N)
    pids = position_ids.reshape(N)
    wt = word_table.reshape(500000, 2 * D)
    pt = pos_table.reshape(MAXPOS // 2, 2 * D)
    out = _emb_ln(ids, pids, wt, pt, ln_weight, ln_bias)
    return out.reshape(*input_ids.shape, D)
